# Initial kernel scaffold; baseline (speedup 1.0000x reference)
import jax
import jax.numpy as jnp
from jax.experimental import pallas as pl


def kernel(x, edge_index, batch, W1, a_src1, a_dst1, b1, W2, a_src2, a_dst2, b2):
    raise NotImplementedError("write your pallas kernel here")


# speedup vs baseline: 2.3512x; 2.3512x over previous
import functools

import jax
import jax.numpy as jnp
from jax.experimental import pallas as pl
from jax.experimental.pallas import tpu as pltpu

_N = 10000
_E = 160000
_D = 256
_WAUG = 384
_RB = 1000
_EB = 1000

_INTERPRET = False


def _matmul_kernel(x_ref, w_ref, a_ref, haug_ref, av_ref):
    h = jnp.dot(x_ref[...], w_ref[...], preferred_element_type=jnp.float32)
    haug_ref[:, :_D] = h
    haug_ref[:, _D:_D + 1] = jnp.ones((h.shape[0], 1), jnp.float32)
    haug_ref[:, _D + 1:] = jnp.zeros((h.shape[0], _WAUG - _D - 1), jnp.float32)
    av_ref[...] = jnp.dot(h, a_ref[...], preferred_element_type=jnp.float32)


def _edge_kernel(src_ref, dst_ref, asrc_ref, adst_ref, haug_ref, av_ref,
                 num_ref):
    @pl.when(pl.program_id(0) == 0)
    def _init():
        e0 = av_ref[:, 0:1] + av_ref[:, 1:2]
        p0 = jnp.exp(jnp.where(e0 >= 0.0, e0, 0.2 * e0))
        num_ref[...] = haug_ref[...] * p0

    def body(j, carry):
        s = src_ref[0, 0, j]
        d = dst_ref[0, 0, j]
        e = asrc_ref[s] + adst_ref[d]
        e = jnp.where(e >= 0.0, e, 0.2 * e)
        pe = jnp.exp(jnp.full((1, 1), e, jnp.float32))
        row = haug_ref[pl.ds(s, 1), :]
        num_ref[pl.ds(d, 1), :] += row * pe
        return carry

    jax.lax.fori_loop(0, _EB, body, 0)


def _combine_kernel(num_ref, b_ref, out_ref, *, relu):
    den = num_ref[:, _D:_D + 1] + 1e-16
    o = num_ref[:, :_D] / den + b_ref[...]
    if relu:
        o = jnp.maximum(o, 0.0)
    out_ref[...] = o


def _gat_layer(x, src3, dst3, W, A, b2d, relu):
    haug, av = pl.pallas_call(
        _matmul_kernel,
        grid=(_N // _RB,),
        in_specs=[
            pl.BlockSpec((_RB, _D), lambda i: (i, 0)),
            pl.BlockSpec((_D, _D), lambda i: (0, 0)),
            pl.BlockSpec((_D, 2), lambda i: (0, 0)),
        ],
        out_specs=[
            pl.BlockSpec((_RB, _WAUG), lambda i: (i, 0)),
            pl.BlockSpec((_RB, 2), lambda i: (i, 0)),
        ],
        out_shape=[
            jax.ShapeDtypeStruct((_N, _WAUG), jnp.float32),
            jax.ShapeDtypeStruct((_N, 2), jnp.float32),
        ],
        interpret=_INTERPRET,
    )(x, W, A)

    asrc = av[:, 0]
    adst = av[:, 1]

    num = pl.pallas_call(
        _edge_kernel,
        grid=(_E // _EB,),
        in_specs=[
            pl.BlockSpec((1, 1, _EB), lambda i: (i, 0, 0),
                         memory_space=pltpu.SMEM),
            pl.BlockSpec((1, 1, _EB), lambda i: (i, 0, 0),
                         memory_space=pltpu.SMEM),
            pl.BlockSpec((_N,), lambda i: (0,), memory_space=pltpu.SMEM),
            pl.BlockSpec((_N,), lambda i: (0,), memory_space=pltpu.SMEM),
            pl.BlockSpec((_N, _WAUG), lambda i: (0, 0)),
            pl.BlockSpec((_N, 2), lambda i: (0, 0)),
        ],
        out_specs=pl.BlockSpec((_N, _WAUG), lambda i: (0, 0)),
        out_shape=jax.ShapeDtypeStruct((_N, _WAUG), jnp.float32),
        interpret=_INTERPRET,
    )(src3, dst3, asrc, adst, haug, av)

    out = pl.pallas_call(
        functools.partial(_combine_kernel, relu=relu),
        grid=(_N // _RB,),
        in_specs=[
            pl.BlockSpec((_RB, _WAUG), lambda i: (i, 0)),
            pl.BlockSpec((1, _D), lambda i: (0, 0)),
        ],
        out_specs=pl.BlockSpec((_RB, _D), lambda i: (i, 0)),
        out_shape=jax.ShapeDtypeStruct((_N, _D), jnp.float32),
        interpret=_INTERPRET,
    )(num, b2d)
    return out


def kernel(x, edge_index, batch, W1, a_src1, a_dst1, b1, W2, a_src2, a_dst2,
           b2):
    src3 = edge_index[0].reshape(_E // _EB, 1, _EB)
    dst3 = edge_index[1].reshape(_E // _EB, 1, _EB)
    A1 = jnp.stack([a_src1, a_dst1], axis=1)
    A2 = jnp.stack([a_src2, a_dst2], axis=1)
    h1 = _gat_layer(x, src3, dst3, W1, A1, b1.reshape(1, _D), relu=True)
    out = _gat_layer(h1, src3, dst3, W2, A2, b2.reshape(1, _D), relu=False)
    return out


# speedup vs baseline: 7.7608x; 3.3008x over previous
import functools

import jax
import jax.numpy as jnp
from jax import lax
from jax.experimental import pallas as pl
from jax.experimental.pallas import tpu as pltpu
from jax.experimental.pallas import tpu_sc as plsc

_N = 10000
_E = 160000
_D = 256
_H = 128
_RB = 1000
_NS = 16
_NPAD = 10112
_RPT = _NPAD // _NS
_EPAD = 163840
_EPT = _EPAD // _NS
_B = 128
_NBLK = _EPT // _B
_CHUNKS = ((0, 128), (128, 128), (256, 128), (384, 128), (512, 120))


def _matmul_kernel(x_ref, w_ref, a_ref, hsc_ref, av_ref):
    h = jnp.dot(x_ref[...], w_ref[...], preferred_element_type=jnp.float32)
    hsc_ref[0] = h[:, :_H]
    hsc_ref[1] = h[:, _H:]
    av_ref[...] = jnp.dot(h, a_ref[...], preferred_element_type=jnp.float32)


def _sc_edge_kernel(hsc_ref, asrc_ref, adst_ref, src_ref, dst_ref,
                    numlo_ref, numhi_ref, den_ref,
                    idx_s, idx_d, gidx, a_s, a_d, p_v, rows,
                    num_sh, den_sh, sem):
    c = lax.axis_index("c")
    s = lax.axis_index("s")
    zero16 = jnp.zeros((16,), jnp.float32)

    def zero_row(i, carry):
        for j in range(8):
            rows[i, pl.ds(j * 16, 16)] = zero16
        return carry

    lax.fori_loop(0, _B, zero_row, 0)
    for j in range(8):
        p_v[pl.ds(j * 16, 16)] = zero16

    r0 = s * _RPT
    for ch, sz in _CHUNKS:
        pltpu.sync_copy(rows.at[pl.ds(0, sz)], num_sh.at[pl.ds(r0 + ch, sz)])
        pltpu.sync_copy(p_v.at[pl.ds(0, sz if sz <= 128 else 128)],
                        den_sh.at[pl.ds(r0 + ch, sz)])
    plsc.subcore_barrier()

    coff = c * _N

    def block_body(b, carry):
        base = s * _EPT + b * _B
        pltpu.sync_copy(src_ref.at[pl.ds(base, _B)], idx_s)
        pltpu.sync_copy(dst_ref.at[pl.ds(base, _B)], idx_d)
        pltpu.async_copy(asrc_ref.at[idx_s], a_s, sem).wait()
        pltpu.async_copy(adst_ref.at[idx_d], a_d, sem).wait()
        for j in range(8):
            sl = pl.ds(j * 16, 16)
            e = a_s[sl] + a_d[sl]
            p_v[sl] = jnp.exp(jnp.maximum(e, 0.2 * e))
            gidx[sl] = idx_s[sl] + coff
        pltpu.async_copy(hsc_ref.at[gidx], rows, sem).wait()

        def scale_row(i, carry2):
            pp = plsc.load_gather(p_v, [jnp.full((16,), i, jnp.int32)])
            for j in range(8):
                sl = pl.ds(j * 16, 16)
                rows[i, sl] = rows[i, sl] * pp
            return carry2

        lax.fori_loop(0, _B, scale_row, 0)
        pltpu.sync_copy(rows, num_sh.at[idx_d], add=True)

        @pl.when(c == 0)
        def _den_add():
            pltpu.sync_copy(p_v, den_sh.at[idx_d], add=True)

        return carry

    lax.fori_loop(0, _NBLK, block_body, 0)
    plsc.subcore_barrier()

    for ch, sz in _CHUNKS:
        pltpu.sync_copy(num_sh.at[pl.ds(r0 + ch, sz)], rows.at[pl.ds(0, sz)])

        @pl.when(c == 0)
        def _write_lo():
            pltpu.sync_copy(rows.at[pl.ds(0, sz)],
                            numlo_ref.at[pl.ds(r0 + ch, sz)])

        @pl.when(c == 1)
        def _write_hi():
            pltpu.sync_copy(rows.at[pl.ds(0, sz)],
                            numhi_ref.at[pl.ds(r0 + ch, sz)])

        @pl.when(c == 0)
        def _write_den():
            pltpu.sync_copy(den_sh.at[pl.ds(r0 + ch, sz)],
                            p_v.at[pl.ds(0, sz)])
            pltpu.sync_copy(p_v.at[pl.ds(0, sz)],
                            den_ref.at[pl.ds(r0 + ch, sz)])


_sc_edge = functools.partial(
    pl.kernel,
    mesh=plsc.VectorSubcoreMesh(core_axis_name="c", subcore_axis_name="s"),
    out_type=[
        jax.ShapeDtypeStruct((_NPAD, _H), jnp.float32),
        jax.ShapeDtypeStruct((_NPAD, _H), jnp.float32),
        jax.ShapeDtypeStruct((_NPAD,), jnp.float32),
    ],
    scratch_types=[
        pltpu.VMEM((_B,), jnp.int32),
        pltpu.VMEM((_B,), jnp.int32),
        pltpu.VMEM((_B,), jnp.int32),
        pltpu.VMEM((_B,), jnp.float32),
        pltpu.VMEM((_B,), jnp.float32),
        pltpu.VMEM((_B,), jnp.float32),
        pltpu.VMEM((_B, _H), jnp.float32),
        pltpu.VMEM_SHARED((_NPAD, _H), jnp.float32),
        pltpu.VMEM_SHARED((_NPAD,), jnp.float32),
        pltpu.SemaphoreType.DMA,
    ],
    compiler_params=pltpu.CompilerParams(needs_layout_passes=False),
)(_sc_edge_kernel)


def _combine_kernel(nlo_ref, nhi_ref, den_ref, av_ref, hlo_ref, hhi_ref,
                    b_ref, out_ref, *, relu):
    e0 = av_ref[:, 0:1] + av_ref[:, 1:2]
    p0 = jnp.exp(jnp.maximum(e0, 0.2 * e0))
    dent = den_ref[...] + p0 + 1e-16
    lo = (nlo_ref[...] + p0 * hlo_ref[0]) / dent + b_ref[:, :_H]
    hi = (nhi_ref[...] + p0 * hhi_ref[0]) / dent + b_ref[:, _H:]
    if relu:
        lo = jnp.maximum(lo, 0.0)
        hi = jnp.maximum(hi, 0.0)
    out_ref[:, :_H] = lo
    out_ref[:, _H:] = hi


def _gat_layer(x, srcp, dstp, W, A, b2d, relu):
    hsc3, av = pl.pallas_call(
        _matmul_kernel,
        grid=(_N // _RB,),
        in_specs=[
            pl.BlockSpec((_RB, _D), lambda i: (i, 0)),
            pl.BlockSpec((_D, _D), lambda i: (0, 0)),
            pl.BlockSpec((_D, 2), lambda i: (0, 0)),
        ],
        out_specs=[
            pl.BlockSpec((2, _RB, _H), lambda i: (0, i, 0)),
            pl.BlockSpec((_RB, 2), lambda i: (i, 0)),
        ],
        out_shape=[
            jax.ShapeDtypeStruct((2, _N, _H), jnp.float32),
            jax.ShapeDtypeStruct((_N, 2), jnp.float32),
        ],
    )(x, W, A)

    hsc2 = hsc3.reshape(2 * _N, _H)
    asrc = av[:, 0]
    adst = av[:, 1]
    num_lo, num_hi, den = _sc_edge(hsc2, asrc, adst, srcp, dstp)

    out = pl.pallas_call(
        functools.partial(_combine_kernel, relu=relu),
        grid=(_N // _RB,),
        in_specs=[
            pl.BlockSpec((_RB, _H), lambda i: (i, 0)),
            pl.BlockSpec((_RB, _H), lambda i: (i, 0)),
            pl.BlockSpec((_RB, 1), lambda i: (i, 0)),
            pl.BlockSpec((_RB, 2), lambda i: (i, 0)),
            pl.BlockSpec((1, _RB, _H), lambda i: (0, i, 0)),
            pl.BlockSpec((1, _RB, _H), lambda i: (1, i, 0)),
            pl.BlockSpec((1, _D), lambda i: (0, 0)),
        ],
        out_specs=pl.BlockSpec((_RB, _D), lambda i: (i, 0)),
        out_shape=jax.ShapeDtypeStruct((_N, _D), jnp.float32),
    )(num_lo, num_hi, den.reshape(_NPAD, 1), av, hsc3, hsc3, b2d)
    return out


def kernel(x, edge_index, batch, W1, a_src1, a_dst1, b1, W2, a_src2, a_dst2,
           b2):
    srcp = jnp.concatenate(
        [edge_index[0], jnp.zeros((_EPAD - _E,), jnp.int32)])
    dstp = jnp.concatenate(
        [edge_index[1], jnp.full((_EPAD - _E,), _N, jnp.int32)])
    A1 = jnp.stack([a_src1, a_dst1], axis=1)
    A2 = jnp.stack([a_src2, a_dst2], axis=1)
    h1 = _gat_layer(x, srcp, dstp, W1, A1, b1.reshape(1, _D), relu=True)
    out = _gat_layer(h1, srcp, dstp, W2, A2, b2.reshape(1, _D), relu=False)
    return out


# speedup vs baseline: 12.2311x; 1.5760x over previous
import functools

import jax
import jax.numpy as jnp
from jax import lax
from jax.experimental import pallas as pl
from jax.experimental.pallas import tpu as pltpu
from jax.experimental.pallas import tpu_sc as plsc

_N = 10000
_E = 160000
_D = 256
_H = 128
_RB = 1000
_NS = 16
_NPAD = 10112
_RPT = _NPAD // _NS
_EPAD = 163840
_EPT = _EPAD // _NS
_B = 128
_NBLK = _EPT // _B
_CHUNKS = ((0, 128), (128, 128), (256, 128), (384, 128), (512, 120))


def _matmul_kernel(x_ref, w_ref, a_ref, hsc_ref, av_ref):
    h = jnp.dot(x_ref[...], w_ref[...], preferred_element_type=jnp.float32)
    hsc_ref[0] = h[:, :_H]
    hsc_ref[1] = h[:, _H:]
    av_ref[...] = jnp.dot(h, a_ref[...], preferred_element_type=jnp.float32)


def _sc_edge_kernel(hsc_ref, asrc_ref, adst_ref, src_ref, dst_ref,
                    numlo_ref, numhi_ref, den_ref,
                    idx_s0, idx_d0, gidx0, a_s0, a_d0, p_v0, rows0,
                    idx_s1, idx_d1, gidx1, a_s1, a_d1, p_v1, rows1,
                    num_sh, den_sh, sem_a0, sem_a1, sem_r0, sem_r1):
    c = lax.axis_index("c")
    s = lax.axis_index("s")
    zero16 = jnp.zeros((16,), jnp.float32)
    buf = ((idx_s0, idx_d0, gidx0, a_s0, a_d0, p_v0, rows0, sem_a0, sem_r0),
           (idx_s1, idx_d1, gidx1, a_s1, a_d1, p_v1, rows1, sem_a1, sem_r1))

    def zero_row(i, carry):
        for j in range(8):
            rows0[i, pl.ds(j * 16, 16)] = zero16
        return carry

    lax.fori_loop(0, _B, zero_row, 0)
    for j in range(8):
        p_v0[pl.ds(j * 16, 16)] = zero16

    r0 = s * _RPT
    for ch, sz in _CHUNKS:
        pltpu.sync_copy(rows0.at[pl.ds(0, sz)], num_sh.at[pl.ds(r0 + ch, sz)])
        pltpu.sync_copy(p_v0.at[pl.ds(0, sz)], den_sh.at[pl.ds(r0 + ch, sz)])
    plsc.subcore_barrier()

    coff = c * _N
    ebase = s * _EPT

    def fetch_and_launch(bn, k):
        idx_s, idx_d, gidx, a_s, a_d, p_v, rows, sem_a, sem_r = buf[k]
        base = ebase + bn * _B
        pltpu.sync_copy(src_ref.at[pl.ds(base, _B)], idx_s)
        pltpu.sync_copy(dst_ref.at[pl.ds(base, _B)], idx_d)
        d1 = pltpu.async_copy(asrc_ref.at[idx_s], a_s, sem_a)
        d2 = pltpu.async_copy(adst_ref.at[idx_d], a_d, sem_a)
        d1.wait()
        d2.wait()
        for j in range(8):
            sl = pl.ds(j * 16, 16)
            e = a_s[sl] + a_d[sl]
            p_v[sl] = jnp.exp(jnp.maximum(e, 0.2 * e))
            gidx[sl] = idx_s[sl] + coff
        pltpu.async_copy(hsc_ref.at[gidx], rows, sem_r)

    def wait_rows(k):
        pltpu.make_async_copy(hsc_ref.at[buf[k][2]], buf[k][6],
                              buf[k][8]).wait()

    def scale(k):
        p_v, rows = buf[k][5], buf[k][6]

        def scale_row(i, carry2):
            i0 = 2 * i
            pp0 = plsc.load_gather(p_v, [jnp.full((16,), i0, jnp.int32)])
            pp1 = plsc.load_gather(p_v, [jnp.full((16,), i0 + 1, jnp.int32)])
            for j in range(8):
                sl = pl.ds(j * 16, 16)
                rows[i0, sl] = rows[i0, sl] * pp0
                rows[i0 + 1, sl] = rows[i0 + 1, sl] * pp1
            return carry2

        lax.fori_loop(0, _B // 2, scale_row, 0)

    def commit(k):
        idx_d, p_v, rows = buf[k][1], buf[k][5], buf[k][6]
        pltpu.sync_copy(rows, num_sh.at[idx_d], add=True)

        @pl.when(c == 0)
        def _den_add():
            pltpu.sync_copy(p_v, den_sh.at[idx_d], add=True)

    fetch_and_launch(0, 0)

    def step(b, cur, nxt):
        bn = b + 1

        @pl.when(bn < _NBLK)
        def _pre():
            fetch_and_launch(bn, nxt)

        wait_rows(cur)
        scale(cur)
        commit(cur)

    def pair_body(g, carry):
        step(2 * g, 0, 1)
        step(2 * g + 1, 1, 0)
        return carry

    lax.fori_loop(0, _NBLK // 2, pair_body, 0)
    plsc.subcore_barrier()

    for ch, sz in _CHUNKS:
        pltpu.sync_copy(num_sh.at[pl.ds(r0 + ch, sz)], rows0.at[pl.ds(0, sz)])

        @pl.when(c == 0)
        def _write_lo():
            pltpu.sync_copy(rows0.at[pl.ds(0, sz)],
                            numlo_ref.at[pl.ds(r0 + ch, sz)])

        @pl.when(c == 1)
        def _write_hi():
            pltpu.sync_copy(rows0.at[pl.ds(0, sz)],
                            numhi_ref.at[pl.ds(r0 + ch, sz)])

        @pl.when(c == 0)
        def _write_den():
            pltpu.sync_copy(den_sh.at[pl.ds(r0 + ch, sz)],
                            p_v0.at[pl.ds(0, sz)])
            pltpu.sync_copy(p_v0.at[pl.ds(0, sz)],
                            den_ref.at[pl.ds(r0 + ch, sz)])


_vmem_set = lambda: [
    pltpu.VMEM((_B,), jnp.int32),
    pltpu.VMEM((_B,), jnp.int32),
    pltpu.VMEM((_B,), jnp.int32),
    pltpu.VMEM((_B,), jnp.float32),
    pltpu.VMEM((_B,), jnp.float32),
    pltpu.VMEM((_B,), jnp.float32),
    pltpu.VMEM((_B, _H), jnp.float32),
]

_sc_edge = functools.partial(
    pl.kernel,
    mesh=plsc.VectorSubcoreMesh(core_axis_name="c", subcore_axis_name="s"),
    out_type=[
        jax.ShapeDtypeStruct((_NPAD, _H), jnp.float32),
        jax.ShapeDtypeStruct((_NPAD, _H), jnp.float32),
        jax.ShapeDtypeStruct((_NPAD,), jnp.float32),
    ],
    scratch_types=(
        _vmem_set() + _vmem_set() + [
            pltpu.VMEM_SHARED((_NPAD, _H), jnp.float32),
            pltpu.VMEM_SHARED((_NPAD,), jnp.float32),
            pltpu.SemaphoreType.DMA,
            pltpu.SemaphoreType.DMA,
            pltpu.SemaphoreType.DMA,
            pltpu.SemaphoreType.DMA,
        ]
    ),
    compiler_params=pltpu.CompilerParams(needs_layout_passes=False),
)(_sc_edge_kernel)


def _combine_kernel(nlo_ref, nhi_ref, den_ref, av_ref, hlo_ref, hhi_ref,
                    b_ref, out_ref, *, relu):
    e0 = av_ref[:, 0:1] + av_ref[:, 1:2]
    p0 = jnp.exp(jnp.maximum(e0, 0.2 * e0))
    dent = den_ref[...] + p0 + 1e-16
    lo = (nlo_ref[...] + p0 * hlo_ref[0]) / dent + b_ref[:, :_H]
    hi = (nhi_ref[...] + p0 * hhi_ref[0]) / dent + b_ref[:, _H:]
    if relu:
        lo = jnp.maximum(lo, 0.0)
        hi = jnp.maximum(hi, 0.0)
    out_ref[:, :_H] = lo
    out_ref[:, _H:] = hi


def _gat_layer(x, srcp, dstp, W, A, b2d, relu):
    hsc3, av = pl.pallas_call(
        _matmul_kernel,
        grid=(_N // _RB,),
        in_specs=[
            pl.BlockSpec((_RB, _D), lambda i: (i, 0)),
            pl.BlockSpec((_D, _D), lambda i: (0, 0)),
            pl.BlockSpec((_D, 2), lambda i: (0, 0)),
        ],
        out_specs=[
            pl.BlockSpec((2, _RB, _H), lambda i: (0, i, 0)),
            pl.BlockSpec((_RB, 2), lambda i: (i, 0)),
        ],
        out_shape=[
            jax.ShapeDtypeStruct((2, _N, _H), jnp.float32),
            jax.ShapeDtypeStruct((_N, 2), jnp.float32),
        ],
    )(x, W, A)

    hsc2 = hsc3.reshape(2 * _N, _H)
    asrc = av[:, 0]
    adst = av[:, 1]
    num_lo, num_hi, den = _sc_edge(hsc2, asrc, adst, srcp, dstp)

    out = pl.pallas_call(
        functools.partial(_combine_kernel, relu=relu),
        grid=(_N // _RB,),
        in_specs=[
            pl.BlockSpec((_RB, _H), lambda i: (i, 0)),
            pl.BlockSpec((_RB, _H), lambda i: (i, 0)),
            pl.BlockSpec((_RB, 1), lambda i: (i, 0)),
            pl.BlockSpec((_RB, 2), lambda i: (i, 0)),
            pl.BlockSpec((1, _RB, _H), lambda i: (0, i, 0)),
            pl.BlockSpec((1, _RB, _H), lambda i: (1, i, 0)),
            pl.BlockSpec((1, _D), lambda i: (0, 0)),
        ],
        out_specs=pl.BlockSpec((_RB, _D), lambda i: (i, 0)),
        out_shape=jax.ShapeDtypeStruct((_N, _D), jnp.float32),
    )(num_lo, num_hi, den.reshape(_NPAD, 1), av, hsc3, hsc3, b2d)
    return out


def kernel(x, edge_index, batch, W1, a_src1, a_dst1, b1, W2, a_src2, a_dst2,
           b2):
    srcp = jnp.concatenate(
        [edge_index[0], jnp.zeros((_EPAD - _E,), jnp.int32)])
    dstp = jnp.concatenate(
        [edge_index[1], jnp.full((_EPAD - _E,), _N, jnp.int32)])
    A1 = jnp.stack([a_src1, a_dst1], axis=1)
    A2 = jnp.stack([a_src2, a_dst2], axis=1)
    h1 = _gat_layer(x, srcp, dstp, W1, A1, b1.reshape(1, _D), relu=True)
    out = _gat_layer(h1, srcp, dstp, W2, A2, b2.reshape(1, _D), relu=False)
    return out


# speedup vs baseline: 12.3501x; 1.0097x over previous
import functools

import jax
import jax.numpy as jnp
from jax import lax
from jax.experimental import pallas as pl
from jax.experimental.pallas import tpu as pltpu
from jax.experimental.pallas import tpu_sc as plsc

_N = 10000
_E = 160000
_D = 256
_H = 128
_RB = 1000
_NS = 16
_NPAD = 10112
_RPT = _NPAD // _NS
_EPAD = 163840
_EPT = _EPAD // _NS
_B = 128
_NBLK = _EPT // _B
_CHUNKS = ((0, 128), (128, 128), (256, 128), (384, 128), (512, 120))


def _matmul_kernel(x_ref, w_ref, a_ref, hsc_ref, av_ref):
    h = jnp.dot(x_ref[...], w_ref[...], preferred_element_type=jnp.float32)
    hsc_ref[0] = h[:, :_H]
    hsc_ref[1] = h[:, _H:]
    av_ref[...] = jnp.dot(h, a_ref[...], preferred_element_type=jnp.float32)


def _sc_edge_kernel(hsc_ref, asrc_ref, adst_ref, src_ref, dst_ref,
                    numlo_ref, numhi_ref, den_ref,
                    idx_s0, idx_d0, gidx0, a_s0, a_d0, p_v0, rows0,
                    idx_s1, idx_d1, gidx1, a_s1, a_d1, p_v1, rows1,
                    zden, num_sh, den_sh,
                    sem_a0, sem_a1, sem_r0, sem_r1, sem_w0, sem_w1):
    c = lax.axis_index("c")
    s = lax.axis_index("s")
    zero16 = jnp.zeros((16,), jnp.float32)
    buf = ((idx_s0, idx_d0, gidx0, a_s0, a_d0, p_v0, rows0,
            sem_a0, sem_r0, sem_w0),
           (idx_s1, idx_d1, gidx1, a_s1, a_d1, p_v1, rows1,
            sem_a1, sem_r1, sem_w1))

    def zero_row(i, carry):
        for j in range(8):
            rows0[i, pl.ds(j * 16, 16)] = zero16
        return carry

    lax.fori_loop(0, 128, zero_row, 0)
    for j in range(8):
        zden[pl.ds(j * 16, 16)] = zero16

    r0 = s * _RPT
    for ch, sz in _CHUNKS:
        pltpu.sync_copy(rows0.at[pl.ds(0, sz)], num_sh.at[pl.ds(r0 + ch, sz)])
        pltpu.sync_copy(zden.at[pl.ds(0, sz)], den_sh.at[pl.ds(r0 + ch, sz)])
    plsc.subcore_barrier()

    coff = c * _N
    ebase = s * _EPT

    def wait_commit(k):
        idx_d, rows = buf[k][1], buf[k][6]
        pltpu.make_async_copy(rows, num_sh.at[idx_d], buf[k][9]).wait()

    def fetch_and_launch(bn, k):
        idx_s, idx_d, gidx, a_s, a_d, p_v, rows, sem_a, sem_r, _ = buf[k]
        base = ebase + bn * _B
        pltpu.sync_copy(src_ref.at[pl.ds(base, _B)], idx_s)
        pltpu.sync_copy(dst_ref.at[pl.ds(base, _B)], idx_d)
        pltpu.async_copy(asrc_ref.at[idx_s], a_s, sem_a)
        pltpu.async_copy(adst_ref.at[idx_d], a_d, sem_a)
        pltpu.make_async_copy(asrc_ref.at[idx_s], a_s, sem_a).wait()
        pltpu.make_async_copy(adst_ref.at[idx_d], a_d, sem_a).wait()
        for j in range(8):
            sl = pl.ds(j * 16, 16)
            e = a_s[sl] + a_d[sl]
            p_v[sl] = jnp.exp(jnp.maximum(e, 0.2 * e))
            gidx[sl] = idx_s[sl] + coff
        pltpu.async_copy(hsc_ref.at[gidx], rows, sem_r)

    def wait_rows(k):
        gidx, rows, sem_r = buf[k][2], buf[k][6], buf[k][8]
        pltpu.make_async_copy(hsc_ref.at[gidx], rows, sem_r).wait()

    def scale(k):
        p_v, rows = buf[k][5], buf[k][6]

        def scale_row(i, carry2):
            i0 = 2 * i
            pp0 = plsc.load_gather(p_v, [jnp.full((16,), i0, jnp.int32)])
            pp1 = plsc.load_gather(p_v, [jnp.full((16,), i0 + 1, jnp.int32)])
            for j in range(8):
                sl = pl.ds(j * 16, 16)
                rows[i0, sl] = rows[i0, sl] * pp0
                rows[i0 + 1, sl] = rows[i0 + 1, sl] * pp1
            return carry2

        lax.fori_loop(0, _B // 2, scale_row, 0)

    def commit(k):
        idx_d, p_v, rows = buf[k][1], buf[k][5], buf[k][6]
        pltpu.async_copy(rows, num_sh.at[idx_d], buf[k][9], add=True)

        @pl.when(c == 0)
        def _den_add():
            pltpu.sync_copy(p_v, den_sh.at[idx_d], add=True)

    fetch_and_launch(0, 0)

    def step(b, cur, nxt):
        bn = b + 1

        @pl.when(bn < _NBLK)
        def _pre():
            @pl.when(bn >= 2)
            def _drain():
                wait_commit(nxt)

            fetch_and_launch(bn, nxt)

        wait_rows(cur)
        scale(cur)
        commit(cur)

    def pair_body(g, carry):
        step(2 * g, 0, 1)
        step(2 * g + 1, 1, 0)
        return carry

    lax.fori_loop(0, _NBLK // 2, pair_body, 0)
    wait_commit(0)
    wait_commit(1)
    plsc.subcore_barrier()

    for ch, sz in _CHUNKS:
        pltpu.sync_copy(num_sh.at[pl.ds(r0 + ch, sz)], rows0.at[pl.ds(0, sz)])

        @pl.when(c == 0)
        def _write_lo():
            pltpu.sync_copy(rows0.at[pl.ds(0, sz)],
                            numlo_ref.at[pl.ds(r0 + ch, sz)])

        @pl.when(c == 1)
        def _write_hi():
            pltpu.sync_copy(rows0.at[pl.ds(0, sz)],
                            numhi_ref.at[pl.ds(r0 + ch, sz)])

        @pl.when(c == 0)
        def _write_den():
            pltpu.sync_copy(den_sh.at[pl.ds(r0 + ch, sz)],
                            zden.at[pl.ds(0, sz)])
            pltpu.sync_copy(zden.at[pl.ds(0, sz)],
                            den_ref.at[pl.ds(r0 + ch, sz)])


_vmem_set = lambda: [
    pltpu.VMEM((_B,), jnp.int32),
    pltpu.VMEM((_B,), jnp.int32),
    pltpu.VMEM((_B,), jnp.int32),
    pltpu.VMEM((_B,), jnp.float32),
    pltpu.VMEM((_B,), jnp.float32),
    pltpu.VMEM((_B,), jnp.float32),
    pltpu.VMEM((_B, _H), jnp.float32),
]

_sc_edge = functools.partial(
    pl.kernel,
    mesh=plsc.VectorSubcoreMesh(core_axis_name="c", subcore_axis_name="s"),
    out_type=[
        jax.ShapeDtypeStruct((_NPAD, _H), jnp.float32),
        jax.ShapeDtypeStruct((_NPAD, _H), jnp.float32),
        jax.ShapeDtypeStruct((_NPAD,), jnp.float32),
    ],
    scratch_types=(
        _vmem_set() + _vmem_set() + [
            pltpu.VMEM((128,), jnp.float32),
            pltpu.VMEM_SHARED((_NPAD, _H), jnp.float32),
            pltpu.VMEM_SHARED((_NPAD,), jnp.float32),
            pltpu.SemaphoreType.DMA,
            pltpu.SemaphoreType.DMA,
            pltpu.SemaphoreType.DMA,
            pltpu.SemaphoreType.DMA,
            pltpu.SemaphoreType.DMA,
            pltpu.SemaphoreType.DMA,
        ]
    ),
    compiler_params=pltpu.CompilerParams(needs_layout_passes=False),
)(_sc_edge_kernel)


def _combine_kernel(nlo_ref, nhi_ref, den_ref, av_ref, hlo_ref, hhi_ref,
                    b_ref, out_ref, *, relu):
    e0 = av_ref[:, 0:1] + av_ref[:, 1:2]
    p0 = jnp.exp(jnp.maximum(e0, 0.2 * e0))
    dent = den_ref[...] + p0 + 1e-16
    lo = (nlo_ref[...] + p0 * hlo_ref[0]) / dent + b_ref[:, :_H]
    hi = (nhi_ref[...] + p0 * hhi_ref[0]) / dent + b_ref[:, _H:]
    if relu:
        lo = jnp.maximum(lo, 0.0)
        hi = jnp.maximum(hi, 0.0)
    out_ref[:, :_H] = lo
    out_ref[:, _H:] = hi


def _gat_layer(x, srcp, dstp, W, A, b2d, relu):
    hsc3, av = pl.pallas_call(
        _matmul_kernel,
        grid=(_N // _RB,),
        in_specs=[
            pl.BlockSpec((_RB, _D), lambda i: (i, 0)),
            pl.BlockSpec((_D, _D), lambda i: (0, 0)),
            pl.BlockSpec((_D, 2), lambda i: (0, 0)),
        ],
        out_specs=[
            pl.BlockSpec((2, _RB, _H), lambda i: (0, i, 0)),
            pl.BlockSpec((_RB, 2), lambda i: (i, 0)),
        ],
        out_shape=[
            jax.ShapeDtypeStruct((2, _N, _H), jnp.float32),
            jax.ShapeDtypeStruct((_N, 2), jnp.float32),
        ],
    )(x, W, A)

    hsc2 = hsc3.reshape(2 * _N, _H)
    asrc = av[:, 0]
    adst = av[:, 1]
    num_lo, num_hi, den = _sc_edge(hsc2, asrc, adst, srcp, dstp)

    out = pl.pallas_call(
        functools.partial(_combine_kernel, relu=relu),
        grid=(_N // _RB,),
        in_specs=[
            pl.BlockSpec((_RB, _H), lambda i: (i, 0)),
            pl.BlockSpec((_RB, _H), lambda i: (i, 0)),
            pl.BlockSpec((_RB, 1), lambda i: (i, 0)),
            pl.BlockSpec((_RB, 2), lambda i: (i, 0)),
            pl.BlockSpec((1, _RB, _H), lambda i: (0, i, 0)),
            pl.BlockSpec((1, _RB, _H), lambda i: (1, i, 0)),
            pl.BlockSpec((1, _D), lambda i: (0, 0)),
        ],
        out_specs=pl.BlockSpec((_RB, _D), lambda i: (i, 0)),
        out_shape=jax.ShapeDtypeStruct((_N, _D), jnp.float32),
    )(num_lo, num_hi, den.reshape(_NPAD, 1), av, hsc3, hsc3, b2d)
    return out


def kernel(x, edge_index, batch, W1, a_src1, a_dst1, b1, W2, a_src2, a_dst2,
           b2):
    srcp = jnp.concatenate(
        [edge_index[0], jnp.zeros((_EPAD - _E,), jnp.int32)])
    dstp = jnp.concatenate(
        [edge_index[1], jnp.full((_EPAD - _E,), _N, jnp.int32)])
    A1 = jnp.stack([a_src1, a_dst1], axis=1)
    A2 = jnp.stack([a_src2, a_dst2], axis=1)
    h1 = _gat_layer(x, srcp, dstp, W1, A1, b1.reshape(1, _D), relu=True)
    out = _gat_layer(h1, srcp, dstp, W2, A2, b2.reshape(1, _D), relu=False)
    return out


# speedup vs baseline: 12.6986x; 1.0282x over previous
import functools

import jax
import jax.numpy as jnp
from jax import lax
from jax.experimental import pallas as pl
from jax.experimental.pallas import tpu as pltpu
from jax.experimental.pallas import tpu_sc as plsc

_N = 10000
_E = 160000
_D = 256
_H = 128
_RB = 1000
_NS = 16
_NPAD = 10112
_RPT = _NPAD // _NS
_EPAD = 163840
_EPT = _EPAD // _NS
_B = 128
_NBLK = _EPT // _B
_CHUNKS = ((0, 128), (128, 128), (256, 128), (384, 128), (512, 120))


def _matmul_kernel(x_ref, w_ref, a_ref, hsc_ref, av_ref):
    h = jnp.dot(x_ref[...], w_ref[...], preferred_element_type=jnp.float32)
    hsc_ref[0] = h[:, :_H]
    hsc_ref[1] = h[:, _H:]
    av_ref[...] = jnp.dot(h, a_ref[...], preferred_element_type=jnp.float32)


def _sc_edge_kernel(hsc_ref, asrc_ref, adst_ref, src_ref, dst_ref,
                    numlo_ref, numhi_ref, den0_ref, den1_ref,
                    idx_s0, idx_d0, gidx0, a_s0, a_d0, p_v0,
                    idx_s1, idx_d1, gidx1, a_s1, a_d1, p_v1,
                    idx_s2, idx_d2, gidx2, a_s2, a_d2, p_v2,
                    rows0, rows1, zden, num_sh, den_sh,
                    sem_a0, sem_a1, sem_a2, sem_r0, sem_r1, sem_w0, sem_w1):
    c = lax.axis_index("c")
    s = lax.axis_index("s")
    zero16 = jnp.zeros((16,), jnp.float32)
    S = ((idx_s0, idx_d0, gidx0, a_s0, a_d0, p_v0, sem_a0),
         (idx_s1, idx_d1, gidx1, a_s1, a_d1, p_v1, sem_a1),
         (idx_s2, idx_d2, gidx2, a_s2, a_d2, p_v2, sem_a2))
    R = ((rows0, sem_r0, sem_w0), (rows1, sem_r1, sem_w1))

    def zero_row(i, carry):
        for j in range(8):
            rows0[i, pl.ds(j * 16, 16)] = zero16
        return carry

    lax.fori_loop(0, _B, zero_row, 0)
    for j in range(8):
        zden[pl.ds(j * 16, 16)] = zero16

    r0 = s * _RPT
    for ch, sz in _CHUNKS:
        pltpu.sync_copy(rows0.at[pl.ds(0, sz)], num_sh.at[pl.ds(r0 + ch, sz)])
        pltpu.sync_copy(zden.at[pl.ds(0, sz)], den_sh.at[pl.ds(r0 + ch, sz)])
    plsc.subcore_barrier()

    coff = c * _N
    ebase = s * _EPT

    def issue_small(bn, si):
        idx_s, idx_d, _, a_s, a_d, _, sem_a = S[si]
        base = ebase + bn * _B
        pltpu.sync_copy(src_ref.at[pl.ds(base, _B)], idx_s)
        pltpu.sync_copy(dst_ref.at[pl.ds(base, _B)], idx_d)
        pltpu.async_copy(asrc_ref.at[idx_s], a_s, sem_a)
        pltpu.async_copy(adst_ref.at[idx_d], a_d, sem_a)

    def wait_commit(ri, si):
        idx_d = S[si][1]
        rows, _, sem_w = R[ri]
        pltpu.make_async_copy(rows, num_sh.at[idx_d], sem_w).wait()

    def launch_rows(b, si, ri, first):
        idx_s, idx_d, gidx, a_s, a_d, p_v, sem_a = S[si]
        rows, sem_r, _ = R[ri]
        pltpu.make_async_copy(asrc_ref.at[idx_s], a_s, sem_a).wait()
        pltpu.make_async_copy(adst_ref.at[idx_d], a_d, sem_a).wait()
        for j in range(8):
            sl = pl.ds(j * 16, 16)
            e = a_s[sl] + a_d[sl]
            p_v[sl] = jnp.exp(jnp.maximum(e, 0.2 * e))
            gidx[sl] = idx_s[sl] + coff
        if not first:
            wait_commit(ri, (si + 1) % 3)
        pltpu.async_copy(hsc_ref.at[gidx], rows, sem_r)

    def wait_rows(si, ri):
        gidx = S[si][2]
        rows, sem_r, _ = R[ri]
        pltpu.make_async_copy(hsc_ref.at[gidx], rows, sem_r).wait()

    def scale(si, ri):
        p_v = S[si][5]
        rows = R[ri][0]

        def scale_row(i, carry2):
            i0 = 2 * i
            pp0 = plsc.load_gather(p_v, [jnp.full((16,), i0, jnp.int32)])
            pp1 = plsc.load_gather(p_v, [jnp.full((16,), i0 + 1, jnp.int32)])
            for j in range(8):
                sl = pl.ds(j * 16, 16)
                rows[i0, sl] = rows[i0, sl] * pp0
                rows[i0 + 1, sl] = rows[i0 + 1, sl] * pp1
            return carry2

        lax.fori_loop(0, _B // 2, scale_row, 0)

    def commit(si, ri, parity):
        idx_d, p_v = S[si][1], S[si][5]
        rows, _, sem_w = R[ri]
        pltpu.async_copy(rows, num_sh.at[idx_d], sem_w, add=True)

        @pl.when(c == parity)
        def _den_add():
            pltpu.sync_copy(p_v, den_sh.at[idx_d], add=True)

    def do_step(b, pos, first_pair=False):
        s_cur, r_cur = pos % 3, pos % 2
        s_nx, r_nx = (pos + 1) % 3, (pos + 1) % 2
        s_n2 = (pos + 2) % 3

        @pl.when(b + 1 < _NBLK)
        def _launch_next():
            launch_rows(b + 1, s_nx, r_nx, first=first_pair)

        @pl.when(b + 2 < _NBLK)
        def _issue_next2():
            issue_small(b + 2, s_n2)

        wait_rows(s_cur, r_cur)
        scale(s_cur, r_cur)
        commit(s_cur, r_cur, parity=pos % 2)

    issue_small(0, 0)
    issue_small(1, 1)
    launch_rows(0, 0, 0, first=True)

    do_step(0, 0, first_pair=True)

    def six_body(g, carry):
        b0 = 6 * g + 1
        for pos in range(6):
            do_step(b0 + pos, (pos + 1) % 6)
        return carry

    lax.fori_loop(0, 12, six_body, 0)
    for b in range(73, _NBLK):
        do_step(b, b % 6)

    wait_commit(0, (_NBLK - 2) % 3)
    wait_commit(1, (_NBLK - 1) % 3)
    plsc.subcore_barrier()

    for ch, sz in _CHUNKS:
        pltpu.sync_copy(num_sh.at[pl.ds(r0 + ch, sz)], rows0.at[pl.ds(0, sz)])

        @pl.when(c == 0)
        def _write_lo():
            pltpu.sync_copy(rows0.at[pl.ds(0, sz)],
                            numlo_ref.at[pl.ds(r0 + ch, sz)])

        @pl.when(c == 1)
        def _write_hi():
            pltpu.sync_copy(rows0.at[pl.ds(0, sz)],
                            numhi_ref.at[pl.ds(r0 + ch, sz)])

        pltpu.sync_copy(den_sh.at[pl.ds(r0 + ch, sz)], zden.at[pl.ds(0, sz)])

        @pl.when(c == 0)
        def _write_den0():
            pltpu.sync_copy(zden.at[pl.ds(0, sz)],
                            den0_ref.at[pl.ds(r0 + ch, sz)])

        @pl.when(c == 1)
        def _write_den1():
            pltpu.sync_copy(zden.at[pl.ds(0, sz)],
                            den1_ref.at[pl.ds(r0 + ch, sz)])


_small_set = lambda: [
    pltpu.VMEM((_B,), jnp.int32),
    pltpu.VMEM((_B,), jnp.int32),
    pltpu.VMEM((_B,), jnp.int32),
    pltpu.VMEM((_B,), jnp.float32),
    pltpu.VMEM((_B,), jnp.float32),
    pltpu.VMEM((_B,), jnp.float32),
]

_sc_edge = functools.partial(
    pl.kernel,
    mesh=plsc.VectorSubcoreMesh(core_axis_name="c", subcore_axis_name="s"),
    out_type=[
        jax.ShapeDtypeStruct((_NPAD, _H), jnp.float32),
        jax.ShapeDtypeStruct((_NPAD, _H), jnp.float32),
        jax.ShapeDtypeStruct((_NPAD,), jnp.float32),
        jax.ShapeDtypeStruct((_NPAD,), jnp.float32),
    ],
    scratch_types=(
        _small_set() + _small_set() + _small_set() + [
            pltpu.VMEM((_B, _H), jnp.float32),
            pltpu.VMEM((_B, _H), jnp.float32),
            pltpu.VMEM((128,), jnp.float32),
            pltpu.VMEM_SHARED((_NPAD, _H), jnp.float32),
            pltpu.VMEM_SHARED((_NPAD,), jnp.float32),
            pltpu.SemaphoreType.DMA,
            pltpu.SemaphoreType.DMA,
            pltpu.SemaphoreType.DMA,
            pltpu.SemaphoreType.DMA,
            pltpu.SemaphoreType.DMA,
            pltpu.SemaphoreType.DMA,
            pltpu.SemaphoreType.DMA,
        ]
    ),
    compiler_params=pltpu.CompilerParams(needs_layout_passes=False),
)(_sc_edge_kernel)


def _combine_kernel(nlo_ref, nhi_ref, den0_ref, den1_ref, av_ref, hlo_ref,
                    hhi_ref, b_ref, out_ref, *, relu):
    e0 = av_ref[:, 0:1] + av_ref[:, 1:2]
    p0 = jnp.exp(jnp.maximum(e0, 0.2 * e0))
    dent = den0_ref[...] + den1_ref[...] + p0 + 1e-16
    lo = (nlo_ref[...] + p0 * hlo_ref[0]) / dent + b_ref[:, :_H]
    hi = (nhi_ref[...] + p0 * hhi_ref[0]) / dent + b_ref[:, _H:]
    if relu:
        lo = jnp.maximum(lo, 0.0)
        hi = jnp.maximum(hi, 0.0)
    out_ref[:, :_H] = lo
    out_ref[:, _H:] = hi


def _gat_layer(x, srcp, dstp, W, A, b2d, relu):
    hsc3, av = pl.pallas_call(
        _matmul_kernel,
        grid=(_N // _RB,),
        in_specs=[
            pl.BlockSpec((_RB, _D), lambda i: (i, 0)),
            pl.BlockSpec((_D, _D), lambda i: (0, 0)),
            pl.BlockSpec((_D, 2), lambda i: (0, 0)),
        ],
        out_specs=[
            pl.BlockSpec((2, _RB, _H), lambda i: (0, i, 0)),
            pl.BlockSpec((_RB, 2), lambda i: (i, 0)),
        ],
        out_shape=[
            jax.ShapeDtypeStruct((2, _N, _H), jnp.float32),
            jax.ShapeDtypeStruct((_N, 2), jnp.float32),
        ],
    )(x, W, A)

    hsc2 = hsc3.reshape(2 * _N, _H)
    asrc = av[:, 0]
    adst = av[:, 1]
    num_lo, num_hi, den0, den1 = _sc_edge(hsc2, asrc, adst, srcp, dstp)

    out = pl.pallas_call(
        functools.partial(_combine_kernel, relu=relu),
        grid=(_N // _RB,),
        in_specs=[
            pl.BlockSpec((_RB, _H), lambda i: (i, 0)),
            pl.BlockSpec((_RB, _H), lambda i: (i, 0)),
            pl.BlockSpec((_RB, 1), lambda i: (i, 0)),
            pl.BlockSpec((_RB, 1), lambda i: (i, 0)),
            pl.BlockSpec((_RB, 2), lambda i: (i, 0)),
            pl.BlockSpec((1, _RB, _H), lambda i: (0, i, 0)),
            pl.BlockSpec((1, _RB, _H), lambda i: (1, i, 0)),
            pl.BlockSpec((1, _D), lambda i: (0, 0)),
        ],
        out_specs=pl.BlockSpec((_RB, _D), lambda i: (i, 0)),
        out_shape=jax.ShapeDtypeStruct((_N, _D), jnp.float32),
    )(num_lo, num_hi, den0.reshape(_NPAD, 1), den1.reshape(_NPAD, 1),
      av, hsc3, hsc3, b2d)
    return out


def kernel(x, edge_index, batch, W1, a_src1, a_dst1, b1, W2, a_src2, a_dst2,
           b2):
    srcp = jnp.concatenate(
        [edge_index[0], jnp.zeros((_EPAD - _E,), jnp.int32)])
    dstp = jnp.concatenate(
        [edge_index[1], jnp.full((_EPAD - _E,), _N, jnp.int32)])
    A1 = jnp.stack([a_src1, a_dst1], axis=1)
    A2 = jnp.stack([a_src2, a_dst2], axis=1)
    h1 = _gat_layer(x, srcp, dstp, W1, A1, b1.reshape(1, _D), relu=True)
    out = _gat_layer(h1, srcp, dstp, W2, A2, b2.reshape(1, _D), relu=False)
    return out


# speedup vs baseline: 13.1609x; 1.0364x over previous
import functools

import jax
import jax.numpy as jnp
from jax import lax
from jax.experimental import pallas as pl
from jax.experimental.pallas import tpu as pltpu
from jax.experimental.pallas import tpu_sc as plsc

_N = 10000
_E = 160000
_D = 256
_H = 128
_RB = 1000
_NS = 16
_NPAD = 10112
_RPT = _NPAD // _NS
_EPAD = 163840
_EPT = _EPAD // _NS
_B = 128
_NBLK = _EPT // _B
_CHUNKS = ((0, 128), (128, 128), (256, 128), (384, 128), (512, 120))


def _matmul_kernel(x_ref, w_ref, a_ref, hsc_ref, av_ref):
    h = jnp.dot(x_ref[...], w_ref[...], preferred_element_type=jnp.float32)
    hsc_ref[0] = h[:, :_H]
    hsc_ref[1] = h[:, _H:]
    av_ref[...] = jnp.dot(h, a_ref[...], preferred_element_type=jnp.float32)


def _sc_edge_kernel(hsc_ref, asrc_ref, adst_ref, src_ref, dst_ref,
                    numlo_ref, numhi_ref, den0_ref, den1_ref,
                    idx_s0, idx_d0, gidx0, a_s0, a_d0, p_v0,
                    idx_s1, idx_d1, gidx1, a_s1, a_d1, p_v1,
                    idx_s2, idx_d2, gidx2, a_s2, a_d2, p_v2,
                    rows0, rows1, zden, num_sh, den_sh,
                    sem_a0, sem_a1, sem_a2, sem_r0, sem_r1, sem_w0, sem_w1):
    c = lax.axis_index("c")
    s = lax.axis_index("s")
    zero16 = jnp.zeros((16,), jnp.float32)
    S = ((idx_s0, idx_d0, gidx0, a_s0, a_d0, p_v0, sem_a0),
         (idx_s1, idx_d1, gidx1, a_s1, a_d1, p_v1, sem_a1),
         (idx_s2, idx_d2, gidx2, a_s2, a_d2, p_v2, sem_a2))
    R = ((rows0, sem_r0, sem_w0), (rows1, sem_r1, sem_w1))

    def zero_row(i, carry):
        for j in range(8):
            rows0[i, pl.ds(j * 16, 16)] = zero16
        return carry

    lax.fori_loop(0, _B, zero_row, 0)
    for j in range(8):
        zden[pl.ds(j * 16, 16)] = zero16

    r0 = s * _RPT
    for ch, sz in _CHUNKS:
        pltpu.sync_copy(rows0.at[pl.ds(0, sz)], num_sh.at[pl.ds(r0 + ch, sz)])
        pltpu.sync_copy(zden.at[pl.ds(0, sz)], den_sh.at[pl.ds(r0 + ch, sz)])
    plsc.subcore_barrier()

    coff = c * _N
    ebase = s * _EPT

    def issue_small(bn, si):
        idx_s, idx_d, _, a_s, a_d, _, sem_a = S[si]
        base = ebase + bn * _B
        pltpu.sync_copy(src_ref.at[pl.ds(base, _B)], idx_s)
        pltpu.sync_copy(dst_ref.at[pl.ds(base, _B)], idx_d)
        pltpu.async_copy(asrc_ref.at[idx_s], a_s, sem_a)
        pltpu.async_copy(adst_ref.at[idx_d], a_d, sem_a)

    def wait_commit(ri, si):
        idx_d = S[si][1]
        rows, _, sem_w = R[ri]
        pltpu.make_async_copy(rows, num_sh.at[idx_d], sem_w).wait()

    def launch_rows(b, si, ri, first):
        idx_s, idx_d, gidx, a_s, a_d, p_v, sem_a = S[si]
        rows, sem_r, _ = R[ri]
        pltpu.make_async_copy(asrc_ref.at[idx_s], a_s, sem_a).wait()
        pltpu.make_async_copy(adst_ref.at[idx_d], a_d, sem_a).wait()
        for j in range(8):
            sl = pl.ds(j * 16, 16)
            e = a_s[sl] + a_d[sl]
            p_v[sl] = jnp.exp(jnp.maximum(e, 0.2 * e))
            gidx[sl] = idx_s[sl] + coff
        if not first:
            wait_commit(ri, (si + 1) % 3)
        pltpu.async_copy(hsc_ref.at[gidx], rows, sem_r)

    def wait_rows(si, ri):
        gidx = S[si][2]
        rows, sem_r, _ = R[ri]
        pltpu.make_async_copy(hsc_ref.at[gidx], rows, sem_r).wait()

    def scale(si, ri):
        p_v = S[si][5]
        rows = R[ri][0]

        def scale_chunk(g, carry2):
            pv = p_v[pl.ds(g * 16, 16)]
            base_row = g * 16
            for l in range(16):
                pp = lax.gather(
                    pv, jnp.full((16, 1), l, jnp.int32),
                    lax.GatherDimensionNumbers(
                        offset_dims=(), collapsed_slice_dims=(0,),
                        start_index_map=(0,)),
                    (1,), mode=lax.GatherScatterMode.PROMISE_IN_BOUNDS)
                r = base_row + l
                for j in range(8):
                    sl = pl.ds(j * 16, 16)
                    rows[r, sl] = rows[r, sl] * pp
            return carry2

        lax.fori_loop(0, _B // 16, scale_chunk, 0)

    def commit(si, ri, parity):
        idx_d, p_v = S[si][1], S[si][5]
        rows, _, sem_w = R[ri]
        pltpu.async_copy(rows, num_sh.at[idx_d], sem_w, add=True)

        @pl.when(c == parity)
        def _den_add():
            pltpu.sync_copy(p_v, den_sh.at[idx_d], add=True)

    def do_step(b, pos, first_pair=False):
        s_cur, r_cur = pos % 3, pos % 2
        s_nx, r_nx = (pos + 1) % 3, (pos + 1) % 2
        s_n2 = (pos + 2) % 3

        @pl.when(b + 1 < _NBLK)
        def _launch_next():
            launch_rows(b + 1, s_nx, r_nx, first=first_pair)

        @pl.when(b + 2 < _NBLK)
        def _issue_next2():
            issue_small(b + 2, s_n2)

        wait_rows(s_cur, r_cur)
        scale(s_cur, r_cur)
        commit(s_cur, r_cur, parity=pos % 2)

    issue_small(0, 0)
    issue_small(1, 1)
    launch_rows(0, 0, 0, first=True)

    do_step(0, 0, first_pair=True)

    def six_body(g, carry):
        b0 = 6 * g + 1
        for pos in range(6):
            do_step(b0 + pos, (pos + 1) % 6)
        return carry

    lax.fori_loop(0, 12, six_body, 0)
    for b in range(73, _NBLK):
        do_step(b, b % 6)

    wait_commit(0, (_NBLK - 2) % 3)
    wait_commit(1, (_NBLK - 1) % 3)
    plsc.subcore_barrier()

    for ch, sz in _CHUNKS:
        pltpu.sync_copy(num_sh.at[pl.ds(r0 + ch, sz)], rows0.at[pl.ds(0, sz)])

        @pl.when(c == 0)
        def _write_lo():
            pltpu.sync_copy(rows0.at[pl.ds(0, sz)],
                            numlo_ref.at[pl.ds(r0 + ch, sz)])

        @pl.when(c == 1)
        def _write_hi():
            pltpu.sync_copy(rows0.at[pl.ds(0, sz)],
                            numhi_ref.at[pl.ds(r0 + ch, sz)])

        pltpu.sync_copy(den_sh.at[pl.ds(r0 + ch, sz)], zden.at[pl.ds(0, sz)])

        @pl.when(c == 0)
        def _write_den0():
            pltpu.sync_copy(zden.at[pl.ds(0, sz)],
                            den0_ref.at[pl.ds(r0 + ch, sz)])

        @pl.when(c == 1)
        def _write_den1():
            pltpu.sync_copy(zden.at[pl.ds(0, sz)],
                            den1_ref.at[pl.ds(r0 + ch, sz)])


_small_set = lambda: [
    pltpu.VMEM((_B,), jnp.int32),
    pltpu.VMEM((_B,), jnp.int32),
    pltpu.VMEM((_B,), jnp.int32),
    pltpu.VMEM((_B,), jnp.float32),
    pltpu.VMEM((_B,), jnp.float32),
    pltpu.VMEM((_B,), jnp.float32),
]

_sc_edge = functools.partial(
    pl.kernel,
    mesh=plsc.VectorSubcoreMesh(core_axis_name="c", subcore_axis_name="s"),
    out_type=[
        jax.ShapeDtypeStruct((_NPAD, _H), jnp.float32),
        jax.ShapeDtypeStruct((_NPAD, _H), jnp.float32),
        jax.ShapeDtypeStruct((_NPAD,), jnp.float32),
        jax.ShapeDtypeStruct((_NPAD,), jnp.float32),
    ],
    scratch_types=(
        _small_set() + _small_set() + _small_set() + [
            pltpu.VMEM((_B, _H), jnp.float32),
            pltpu.VMEM((_B, _H), jnp.float32),
            pltpu.VMEM((128,), jnp.float32),
            pltpu.VMEM_SHARED((_NPAD, _H), jnp.float32),
            pltpu.VMEM_SHARED((_NPAD,), jnp.float32),
            pltpu.SemaphoreType.DMA,
            pltpu.SemaphoreType.DMA,
            pltpu.SemaphoreType.DMA,
            pltpu.SemaphoreType.DMA,
            pltpu.SemaphoreType.DMA,
            pltpu.SemaphoreType.DMA,
            pltpu.SemaphoreType.DMA,
        ]
    ),
    compiler_params=pltpu.CompilerParams(needs_layout_passes=False),
)(_sc_edge_kernel)


def _combine_kernel(nlo_ref, nhi_ref, den0_ref, den1_ref, av_ref, hlo_ref,
                    hhi_ref, b_ref, out_ref, *, relu):
    e0 = av_ref[:, 0:1] + av_ref[:, 1:2]
    p0 = jnp.exp(jnp.maximum(e0, 0.2 * e0))
    dent = den0_ref[...] + den1_ref[...] + p0 + 1e-16
    lo = (nlo_ref[...] + p0 * hlo_ref[0]) / dent + b_ref[:, :_H]
    hi = (nhi_ref[...] + p0 * hhi_ref[0]) / dent + b_ref[:, _H:]
    if relu:
        lo = jnp.maximum(lo, 0.0)
        hi = jnp.maximum(hi, 0.0)
    out_ref[:, :_H] = lo
    out_ref[:, _H:] = hi


def _gat_layer(x, srcp, dstp, W, A, b2d, relu):
    hsc3, av = pl.pallas_call(
        _matmul_kernel,
        grid=(_N // _RB,),
        in_specs=[
            pl.BlockSpec((_RB, _D), lambda i: (i, 0)),
            pl.BlockSpec((_D, _D), lambda i: (0, 0)),
            pl.BlockSpec((_D, 2), lambda i: (0, 0)),
        ],
        out_specs=[
            pl.BlockSpec((2, _RB, _H), lambda i: (0, i, 0)),
            pl.BlockSpec((_RB, 2), lambda i: (i, 0)),
        ],
        out_shape=[
            jax.ShapeDtypeStruct((2, _N, _H), jnp.float32),
            jax.ShapeDtypeStruct((_N, 2), jnp.float32),
        ],
    )(x, W, A)

    hsc2 = hsc3.reshape(2 * _N, _H)
    asrc = av[:, 0]
    adst = av[:, 1]
    num_lo, num_hi, den0, den1 = _sc_edge(hsc2, asrc, adst, srcp, dstp)

    out = pl.pallas_call(
        functools.partial(_combine_kernel, relu=relu),
        grid=(_N // _RB,),
        in_specs=[
            pl.BlockSpec((_RB, _H), lambda i: (i, 0)),
            pl.BlockSpec((_RB, _H), lambda i: (i, 0)),
            pl.BlockSpec((_RB, 1), lambda i: (i, 0)),
            pl.BlockSpec((_RB, 1), lambda i: (i, 0)),
            pl.BlockSpec((_RB, 2), lambda i: (i, 0)),
            pl.BlockSpec((1, _RB, _H), lambda i: (0, i, 0)),
            pl.BlockSpec((1, _RB, _H), lambda i: (1, i, 0)),
            pl.BlockSpec((1, _D), lambda i: (0, 0)),
        ],
        out_specs=pl.BlockSpec((_RB, _D), lambda i: (i, 0)),
        out_shape=jax.ShapeDtypeStruct((_N, _D), jnp.float32),
    )(num_lo, num_hi, den0.reshape(_NPAD, 1), den1.reshape(_NPAD, 1),
      av, hsc3, hsc3, b2d)
    return out


def kernel(x, edge_index, batch, W1, a_src1, a_dst1, b1, W2, a_src2, a_dst2,
           b2):
    srcp = jnp.concatenate(
        [edge_index[0], jnp.zeros((_EPAD - _E,), jnp.int32)])
    dstp = jnp.concatenate(
        [edge_index[1], jnp.full((_EPAD - _E,), _N, jnp.int32)])
    A1 = jnp.stack([a_src1, a_dst1], axis=1)
    A2 = jnp.stack([a_src2, a_dst2], axis=1)
    h1 = _gat_layer(x, srcp, dstp, W1, A1, b1.reshape(1, _D), relu=True)
    out = _gat_layer(h1, srcp, dstp, W2, A2, b2.reshape(1, _D), relu=False)
    return out
